# baseline (speedup 1.0000x reference)
import functools

import jax
import jax.numpy as jnp
from jax import lax
from jax.experimental import pallas as pl
from jax.experimental.pallas import tpu as pltpu
from jax.experimental.pallas import tpu_sc as plsc

HF = 128
_BN = 2000
_BE = 4000
_IW = 125


def _mm_t(x, w):
    return lax.dot_general(x, w, (((1,), (1,)), ((), ())),
                           preferred_element_type=jnp.float32)


def _lstm_math(gates, c):
    i, f, g, o = jnp.split(gates, 4, axis=1)
    c2 = jax.nn.sigmoid(f) * c + jax.nn.sigmoid(i) * jnp.tanh(g)
    h2 = jax.nn.sigmoid(o) * jnp.tanh(c2)
    return h2, c2


def _node0_body(x_ref, wih_ref, nb_ref, pw_ref, pb_ref, ws_ref, wd_ref,
                nh_ref, nc_ref, out_ref, ps_ref, pd_ref):
    gates = _mm_t(x_ref[...], wih_ref[...]) + nb_ref[...]
    h2, c2 = _lstm_math(gates, 0.0)
    nh_ref[...] = h2
    nc_ref[...] = c2
    out_ref[...] = _mm_t(h2, pw_ref[...]) + pb_ref[...]
    ps_ref[...] = _mm_t(h2, ws_ref[...]).astype(jnp.bfloat16)
    pd_ref[...] = _mm_t(h2, wd_ref[...]).astype(jnp.bfloat16)


def _node_body(s0_ref, s1_ref, deg_ref, nh_ref, nc_ref, out_ref,
               wnh_ref, nmb_ref, wih_ref, whh_ref, nb_ref, pw_ref, pb_ref,
               ws_ref, wd_ref,
               nh2_ref, nc2_ref, out2_ref, ps_ref, pd_ref):
    nh = nh_ref[...]
    s = (s0_ref[0] + s1_ref[0]) / jnp.maximum(deg_ref[...], 1.0)
    x = jax.nn.leaky_relu(s + _mm_t(nh, wnh_ref[...]) + nmb_ref[...], 0.01)
    gates = (_mm_t(x, wih_ref[...]) + _mm_t(nh, whh_ref[...]) + nb_ref[...])
    h2, c2 = _lstm_math(gates, nc_ref[...])
    nh2_ref[...] = h2
    nc2_ref[...] = c2
    out2_ref[...] = out_ref[...] + _mm_t(h2, pw_ref[...]) + pb_ref[...]
    ps_ref[...] = _mm_t(h2, ws_ref[...]).astype(jnp.bfloat16)
    pd_ref[...] = _mm_t(h2, wd_ref[...]).astype(jnp.bfloat16)


def _edge0_body(ef_ref, f1w_ref, f1b_ref, wih_ref, eb_ref, wmm_ref,
                eh_ref, ec_ref, qe_ref):
    x0 = _mm_t(ef_ref[...], f1w_ref[...]) + f1b_ref[...]
    gates = _mm_t(x0, wih_ref[...]) + eb_ref[...]
    h2, c2 = _lstm_math(gates, 0.0)
    eh_ref[...] = h2.astype(jnp.bfloat16)
    ec_ref[...] = c2.astype(jnp.bfloat16)
    qe_ref[...] = _mm_t(h2, wmm_ref[...])


def _edge_body(gs_ref, gd_ref, eh_ref, ec_ref,
               we_ref, emb_ref, wih_ref, whh_ref, eb_ref, wmm_ref,
               eh2_ref, ec2_ref, qe_ref):
    ehb = eh_ref[...]
    x = jax.nn.leaky_relu(
        gs_ref[...].astype(jnp.float32) + gd_ref[...].astype(jnp.float32) +
        _mm_t(ehb, we_ref[...]) + emb_ref[...], 0.01)
    gates = (_mm_t(x.astype(jnp.bfloat16), wih_ref[...]) +
             _mm_t(ehb, whh_ref[...]) + eb_ref[...])
    h2, c2 = _lstm_math(gates, ec_ref[...].astype(jnp.float32))
    h2b = h2.astype(jnp.bfloat16)
    eh2_ref[...] = h2b
    ec2_ref[...] = c2.astype(jnp.bfloat16)
    qe_ref[...] = _mm_t(h2b, wmm_ref[...])


def _full_spec(a):
    nd = a.ndim
    return pl.BlockSpec(a.shape, lambda i, _nd=nd: (0,) * _nd)


def _row_spec(block_rows, cols, off=0):
    return pl.BlockSpec((block_rows, cols), lambda i, _o=off: (i + _o, 0))


def _node0_call(node_feat, wih, nb, pw, pb, ws, wd):
    n = node_feat.shape[0]
    f32 = jnp.float32
    return pl.pallas_call(
        _node0_body,
        grid=(n // _BN,),
        in_specs=[_row_spec(_BN, node_feat.shape[1])] +
                 [_full_spec(a) for a in (wih, nb, pw, pb, ws, wd)],
        out_specs=[_row_spec(_BN, HF), _row_spec(_BN, HF), _row_spec(_BN, 4),
                   _row_spec(_BN, 20), _row_spec(_BN, 20)],
        out_shape=[jax.ShapeDtypeStruct((n, HF), f32),
                   jax.ShapeDtypeStruct((n, HF), f32),
                   jax.ShapeDtypeStruct((n, 4), f32),
                   jax.ShapeDtypeStruct((n, 20), jnp.bfloat16),
                   jax.ShapeDtypeStruct((n, 20), jnp.bfloat16)],
    )(node_feat, wih, nb, pw, pb, ws, wd)


def _node_call(sp, deg, nh, nc, out, wnh, nmb, wih, whh, nb, pw, pb,
               ws, wd):
    n = nh.shape[0]
    f32 = jnp.float32
    sp_spec0 = pl.BlockSpec((1, _BN, 16), lambda i: (0, i, 0))
    sp_spec1 = pl.BlockSpec((1, _BN, 16), lambda i: (1, i, 0))
    return pl.pallas_call(
        _node_body,
        grid=(n // _BN,),
        in_specs=[sp_spec0, sp_spec1, _row_spec(_BN, 1),
                  _row_spec(_BN, HF), _row_spec(_BN, HF), _row_spec(_BN, 4)] +
                 [_full_spec(a) for a in (wnh, nmb, wih, whh, nb, pw, pb,
                                          ws, wd)],
        out_specs=[_row_spec(_BN, HF), _row_spec(_BN, HF), _row_spec(_BN, 4),
                   _row_spec(_BN, 20), _row_spec(_BN, 20)],
        out_shape=[jax.ShapeDtypeStruct((n, HF), f32),
                   jax.ShapeDtypeStruct((n, HF), f32),
                   jax.ShapeDtypeStruct((n, 4), f32),
                   jax.ShapeDtypeStruct((n, 20), jnp.bfloat16),
                   jax.ShapeDtypeStruct((n, 20), jnp.bfloat16)],
    )(sp, sp, deg, nh, nc, out, wnh, nmb, wih, whh, nb, pw, pb, ws, wd)


def _edge0_call(edge_feat, f1w, f1b, wih, eb, wmm):
    e = edge_feat.shape[0]
    f32 = jnp.float32
    return pl.pallas_call(
        _edge0_body,
        grid=(e // _BE,),
        in_specs=[_row_spec(_BE, edge_feat.shape[1])] +
                 [_full_spec(a) for a in (f1w, f1b, wih, eb, wmm)],
        out_specs=[_row_spec(_BE, HF), _row_spec(_BE, HF), _row_spec(_BE, 16)],
        out_shape=[jax.ShapeDtypeStruct((e, HF), jnp.bfloat16),
                   jax.ShapeDtypeStruct((e, HF), jnp.bfloat16),
                   jax.ShapeDtypeStruct((e, 16), f32)],
    )(edge_feat, f1w, f1b, wih, eb, wmm)


def _edge_call(g, eh, ec, we, emb, wih, whh, eb, wmm):
    e = eh.shape[0]
    f32 = jnp.float32
    nbe = e // _BE
    return pl.pallas_call(
        _edge_body,
        grid=(nbe,),
        in_specs=[_row_spec(_BE, 20), _row_spec(_BE, 20, off=nbe),
                  _row_spec(_BE, HF), _row_spec(_BE, HF)] +
                 [_full_spec(a) for a in (we, emb, wih, whh, eb, wmm)],
        out_specs=[_row_spec(_BE, HF), _row_spec(_BE, HF), _row_spec(_BE, 16)],
        out_shape=[jax.ShapeDtypeStruct((e, HF), jnp.bfloat16),
                   jax.ShapeDtypeStruct((e, HF), jnp.bfloat16),
                   jax.ShapeDtypeStruct((e, 16), f32)],
    )(g, g, eh, ec, we, emb, wih, whh, eb, wmm)


_NW = 32
_GCH = 8


def _sc_scatter_gather(q, dst2d, zeros, ps, pd, src2d):
    n = zeros.shape[0]
    e = q.shape[0]
    n_rows = dst2d.shape[0]
    srows_per_w = n_rows // _NW
    qrows_per_w = srows_per_w * _IW
    grows_per_w = n_rows // (_NW // 2)
    gout_per_w = grows_per_w * _IW
    n_ch = grows_per_w // _GCH
    nps = n // 16
    mesh = plsc.VectorSubcoreMesh(core_axis_name="c", subcore_axis_name="s")

    @functools.partial(
        pl.kernel, mesh=mesh,
        out_type=[jax.ShapeDtypeStruct((2, n, 16), jnp.float32),
                  jax.ShapeDtypeStruct((2 * e, 20), jnp.bfloat16)],
        compiler_params=pltpu.CompilerParams(use_tc_tiling_on_sc=False),
        scratch_types=[pltpu.VMEM((srows_per_w, _IW), jnp.int32),
                       pltpu.VMEM((qrows_per_w, 16), jnp.float32),
                       pltpu.VMEM_SHARED((n, 16), jnp.float32),
                       pltpu.VMEM((grows_per_w, _IW), jnp.int32),
                       pltpu.VMEM((_GCH * _IW, 20), jnp.bfloat16),
                       pltpu.SemaphoreType.DMA])
    def k(q_hbm, dst_hbm, z_hbm, ps_hbm, pd_hbm, src_hbm,
          sp_hbm, g_hbm, dst_v, q_v, acc, idx_v, rows_v, sem):
        cid = lax.axis_index("c")
        sid = lax.axis_index("s")
        wid = sid * 2 + cid
        lds = [pltpu.async_copy(z_hbm.at[pl.ds(sid * nps, nps)],
                                acc.at[pl.ds(sid * nps, nps)], sem),
               pltpu.async_copy(dst_hbm.at[pl.ds(wid * srows_per_w,
                                                 srows_per_w)], dst_v, sem),
               pltpu.async_copy(q_hbm.at[pl.ds(wid * qrows_per_w,
                                               qrows_per_w)], q_v, sem)]
        for c in lds:
            c.wait()
        plsc.subcore_barrier()

        def sbody(t, carry):
            cps = []
            for j in range(8):
                r = t * 8 + j
                cps.append(pltpu.async_copy(q_v.at[pl.ds(r * _IW, _IW)],
                                            acc.at[dst_v.at[r]], sem,
                                            add=True))
            for c in cps:
                c.wait()
            return carry

        lax.fori_loop(0, srows_per_w // 8, sbody, 0)
        plsc.subcore_barrier()
        pltpu.sync_copy(acc.at[pl.ds(sid * nps, nps)],
                        sp_hbm.at[cid, pl.ds(sid * nps, nps)])

        obase = wid * gout_per_w

        def gather_from(tbl_hbm, ix_hbm, ibase):
            pltpu.sync_copy(ix_hbm.at[pl.ds(ibase, grows_per_w)], idx_v)

            def chunk(t, carry):
                cps = []
                for j in range(_GCH):
                    cps.append(pltpu.async_copy(
                        tbl_hbm.at[idx_v.at[t * _GCH + j]],
                        rows_v.at[pl.ds(j * _IW, _IW)], sem))
                for c in cps:
                    c.wait()
                pltpu.sync_copy(
                    rows_v,
                    g_hbm.at[pl.ds(obase + t * (_GCH * _IW), _GCH * _IW)])
                return carry

            lax.fori_loop(0, n_ch, chunk, 0)

        @pl.when(wid < 16)
        def _():
            gather_from(ps_hbm, src_hbm, wid * grows_per_w)

        @pl.when(wid >= 16)
        def _():
            gather_from(pd_hbm, dst_hbm, (wid - 16) * grows_per_w)

    return k(q, dst2d, zeros, ps, pd, src2d)


def _sc_scatter(q, dst2d, zeros):
    n = zeros.shape[0]
    n_rows = dst2d.shape[0]
    rows_per_w = n_rows // _NW
    qrows_per_w = rows_per_w * _IW
    nps = n // 16
    mesh = plsc.VectorSubcoreMesh(core_axis_name="c", subcore_axis_name="s")

    @functools.partial(
        pl.kernel, mesh=mesh,
        out_type=jax.ShapeDtypeStruct((2, n, 16), jnp.float32),
        compiler_params=pltpu.CompilerParams(use_tc_tiling_on_sc=False),
        scratch_types=[pltpu.VMEM((rows_per_w, _IW), jnp.int32),
                       pltpu.VMEM((qrows_per_w, 16), jnp.float32),
                       pltpu.VMEM_SHARED((n, 16), jnp.float32),
                       pltpu.SemaphoreType.DMA])
    def k(q_hbm, dst_hbm, z_hbm, out_hbm, dst_v, q_v, acc, sem):
        cid = lax.axis_index("c")
        sid = lax.axis_index("s")
        wid = sid * 2 + cid
        lds = [pltpu.async_copy(z_hbm.at[pl.ds(sid * nps, nps)],
                                acc.at[pl.ds(sid * nps, nps)], sem),
               pltpu.async_copy(dst_hbm.at[pl.ds(wid * rows_per_w,
                                                 rows_per_w)], dst_v, sem),
               pltpu.async_copy(q_hbm.at[pl.ds(wid * qrows_per_w,
                                               qrows_per_w)], q_v, sem)]
        for c in lds:
            c.wait()
        plsc.subcore_barrier()

        def body(t, carry):
            cps = []
            for j in range(8):
                r = t * 8 + j
                cps.append(pltpu.async_copy(q_v.at[pl.ds(r * _IW, _IW)],
                                            acc.at[dst_v.at[r]], sem,
                                            add=True))
            for c in cps:
                c.wait()
            return carry

        lax.fori_loop(0, rows_per_w // 8, body, 0)
        plsc.subcore_barrier()
        pltpu.sync_copy(acc.at[pl.ds(sid * nps, nps)],
                        out_hbm.at[cid, pl.ds(sid * nps, nps)])

    return k(q, dst2d, zeros)


def kernel(node_feat, edge_feat, edge_index, fc1_W, fc1_b,
           n_Wih, n_Whh, n_bih, n_bhh,
           e_Wih, e_Whh, e_bih, e_bhh,
           node_mpn_W, node_mpn_b, edge_mpn_W, edge_mpn_b,
           pred_W, pred_b):
    n = node_feat.shape[0]
    e = edge_feat.shape[0]
    niter = pred_b.shape[0]
    f32 = jnp.float32

    src = edge_index[0]
    dst = edge_index[1]
    src2d = src.reshape(-1, _IW)
    dst2d = dst.reshape(-1, _IW)

    nb = (n_bih + n_bhh).reshape(1, -1)
    eb = (e_bih + e_bhh).reshape(1, -1)
    emb = edge_mpn_b.reshape(1, -1)
    nmb = node_mpn_b.reshape(1, -1)
    f1b = fc1_b.reshape(1, -1)
    ws = edge_mpn_W[:, :HF]
    we = edge_mpn_W[:, HF:2 * HF]
    wd = edge_mpn_W[:, 2 * HF:]
    wmm = node_mpn_W[:, :HF]
    wnh = node_mpn_W[:, HF:]
    bf16 = jnp.bfloat16
    we_b = we.astype(bf16)
    e_Wih_b = e_Wih.astype(bf16)
    e_Whh_b = e_Whh.astype(bf16)
    wmm_b = wmm.astype(bf16)

    n_pad = ((n + 127) // 128) * 128
    zeros_n = jnp.zeros((n_pad, 16), f32)
    deg_p = _sc_scatter(jnp.ones((e, 16), f32), dst2d, zeros_n)
    deg = deg_p[0, :n, :1] + deg_p[1, :n, :1]

    nh, nc, out, ps, pd = _node0_call(
        node_feat, n_Wih, nb, pred_W[0], pred_b[0].reshape(1, -1), ws, wd)
    eh, ec, qe = _edge0_call(edge_feat, fc1_W, f1b, e_Wih, eb, wmm)

    for it in range(1, niter):
        last = it == niter - 1
        if last:
            sp = _sc_scatter(qe, dst2d, zeros_n)
        else:
            sp, g = _sc_scatter_gather(qe, dst2d, zeros_n, ps, pd, src2d)
        nh, nc, out, ps, pd = _node_call(
            sp, deg, nh, nc, out, wnh, nmb, n_Wih, n_Whh, nb,
            pred_W[it], pred_b[it].reshape(1, -1), ws, wd)
        if not last:
            eh, ec, qe = _edge_call(g, eh, ec, we_b, emb, e_Wih_b, e_Whh_b,
                                    eb, wmm_b)

    return out


# speedup vs baseline: 1.0096x; 1.0096x over previous
import functools

import jax
import jax.numpy as jnp
from jax import lax
from jax.experimental import pallas as pl
from jax.experimental.pallas import tpu as pltpu
from jax.experimental.pallas import tpu_sc as plsc

HF = 128
_BN = 2000
_BE = 4000
_IW = 125


def _mm_t(x, w):
    return lax.dot_general(x, w, (((1,), (1,)), ((), ())),
                           preferred_element_type=jnp.float32)


def _lstm_math(gates, c):
    i, f, g, o = jnp.split(gates, 4, axis=1)
    c2 = jax.nn.sigmoid(f) * c + jax.nn.sigmoid(i) * jnp.tanh(g)
    h2 = jax.nn.sigmoid(o) * jnp.tanh(c2)
    return h2, c2


def _node0_body(x_ref, wih_ref, nb_ref, pw_ref, pb_ref, ws_ref, wd_ref,
                nh_ref, nc_ref, out_ref, ps_ref, pd_ref):
    gates = _mm_t(x_ref[...], wih_ref[...]) + nb_ref[...]
    h2, c2 = _lstm_math(gates, 0.0)
    nh_ref[...] = h2
    nc_ref[...] = c2
    out_ref[...] = _mm_t(h2, pw_ref[...]) + pb_ref[...]
    ps_ref[...] = _mm_t(h2, ws_ref[...])
    pd_ref[...] = _mm_t(h2, wd_ref[...])


def _node_body(s0_ref, s1_ref, deg_ref, nh_ref, nc_ref, out_ref,
               wnh_ref, nmb_ref, wih_ref, whh_ref, nb_ref, pw_ref, pb_ref,
               ws_ref, wd_ref,
               nh2_ref, nc2_ref, out2_ref, ps_ref, pd_ref):
    nh = nh_ref[...]
    s = (s0_ref[0] + s1_ref[0]) / jnp.maximum(deg_ref[...], 1.0)
    x = jax.nn.leaky_relu(s + _mm_t(nh, wnh_ref[...]) + nmb_ref[...], 0.01)
    gates = (_mm_t(x, wih_ref[...]) + _mm_t(nh, whh_ref[...]) + nb_ref[...])
    h2, c2 = _lstm_math(gates, nc_ref[...])
    nh2_ref[...] = h2
    nc2_ref[...] = c2
    out2_ref[...] = out_ref[...] + _mm_t(h2, pw_ref[...]) + pb_ref[...]
    ps_ref[...] = _mm_t(h2, ws_ref[...])
    pd_ref[...] = _mm_t(h2, wd_ref[...])


def _edge0_body(ef_ref, f1w_ref, f1b_ref, wih_ref, eb_ref, wmm_ref,
                eh_ref, ec_ref, qe_ref):
    x0 = _mm_t(ef_ref[...], f1w_ref[...]) + f1b_ref[...]
    gates = _mm_t(x0, wih_ref[...]) + eb_ref[...]
    h2, c2 = _lstm_math(gates, 0.0)
    eh_ref[...] = h2.astype(jnp.bfloat16)
    ec_ref[...] = c2.astype(jnp.bfloat16)
    qe_ref[...] = _mm_t(h2, wmm_ref[...])


def _edge_body(gs_ref, gd_ref, eh_ref, ec_ref,
               we_ref, emb_ref, wih_ref, whh_ref, eb_ref, wmm_ref,
               eh2_ref, ec2_ref, qe_ref):
    ehb = eh_ref[...]
    x = jax.nn.leaky_relu(
        gs_ref[...] + gd_ref[...] + _mm_t(ehb, we_ref[...]) + emb_ref[...],
        0.01)
    gates = (_mm_t(x.astype(jnp.bfloat16), wih_ref[...]) +
             _mm_t(ehb, whh_ref[...]) + eb_ref[...])
    h2, c2 = _lstm_math(gates, ec_ref[...].astype(jnp.float32))
    h2b = h2.astype(jnp.bfloat16)
    eh2_ref[...] = h2b
    ec2_ref[...] = c2.astype(jnp.bfloat16)
    qe_ref[...] = _mm_t(h2b, wmm_ref[...])


def _full_spec(a):
    nd = a.ndim
    return pl.BlockSpec(a.shape, lambda i, _nd=nd: (0,) * _nd)


def _row_spec(block_rows, cols, off=0):
    return pl.BlockSpec((block_rows, cols), lambda i, _o=off: (i + _o, 0))


def _node0_call(node_feat, wih, nb, pw, pb, ws, wd):
    n = node_feat.shape[0]
    f32 = jnp.float32
    return pl.pallas_call(
        _node0_body,
        grid=(n // _BN,),
        in_specs=[_row_spec(_BN, node_feat.shape[1])] +
                 [_full_spec(a) for a in (wih, nb, pw, pb, ws, wd)],
        out_specs=[_row_spec(_BN, HF), _row_spec(_BN, HF), _row_spec(_BN, 4),
                   _row_spec(_BN, 20), _row_spec(_BN, 20)],
        out_shape=[jax.ShapeDtypeStruct((n, HF), f32),
                   jax.ShapeDtypeStruct((n, HF), f32),
                   jax.ShapeDtypeStruct((n, 4), f32),
                   jax.ShapeDtypeStruct((n, 20), f32),
                   jax.ShapeDtypeStruct((n, 20), f32)],
    )(node_feat, wih, nb, pw, pb, ws, wd)


def _node_call(sp, deg, nh, nc, out, wnh, nmb, wih, whh, nb, pw, pb,
               ws, wd):
    n = nh.shape[0]
    f32 = jnp.float32
    sp_spec0 = pl.BlockSpec((1, _BN, 16), lambda i: (0, i, 0))
    sp_spec1 = pl.BlockSpec((1, _BN, 16), lambda i: (1, i, 0))
    return pl.pallas_call(
        _node_body,
        grid=(n // _BN,),
        in_specs=[sp_spec0, sp_spec1, _row_spec(_BN, 1),
                  _row_spec(_BN, HF), _row_spec(_BN, HF), _row_spec(_BN, 4)] +
                 [_full_spec(a) for a in (wnh, nmb, wih, whh, nb, pw, pb,
                                          ws, wd)],
        out_specs=[_row_spec(_BN, HF), _row_spec(_BN, HF), _row_spec(_BN, 4),
                   _row_spec(_BN, 20), _row_spec(_BN, 20)],
        out_shape=[jax.ShapeDtypeStruct((n, HF), f32),
                   jax.ShapeDtypeStruct((n, HF), f32),
                   jax.ShapeDtypeStruct((n, 4), f32),
                   jax.ShapeDtypeStruct((n, 20), f32),
                   jax.ShapeDtypeStruct((n, 20), f32)],
    )(sp, sp, deg, nh, nc, out, wnh, nmb, wih, whh, nb, pw, pb, ws, wd)


def _edge0_call(edge_feat, f1w, f1b, wih, eb, wmm):
    e = edge_feat.shape[0]
    f32 = jnp.float32
    return pl.pallas_call(
        _edge0_body,
        grid=(e // _BE,),
        in_specs=[_row_spec(_BE, edge_feat.shape[1])] +
                 [_full_spec(a) for a in (f1w, f1b, wih, eb, wmm)],
        out_specs=[_row_spec(_BE, HF), _row_spec(_BE, HF), _row_spec(_BE, 16)],
        out_shape=[jax.ShapeDtypeStruct((e, HF), jnp.bfloat16),
                   jax.ShapeDtypeStruct((e, HF), jnp.bfloat16),
                   jax.ShapeDtypeStruct((e, 16), f32)],
    )(edge_feat, f1w, f1b, wih, eb, wmm)


def _edge_call(g, eh, ec, we, emb, wih, whh, eb, wmm):
    e = eh.shape[0]
    f32 = jnp.float32
    nbe = e // _BE
    return pl.pallas_call(
        _edge_body,
        grid=(nbe,),
        in_specs=[_row_spec(_BE, 20), _row_spec(_BE, 20, off=nbe),
                  _row_spec(_BE, HF), _row_spec(_BE, HF)] +
                 [_full_spec(a) for a in (we, emb, wih, whh, eb, wmm)],
        out_specs=[_row_spec(_BE, HF), _row_spec(_BE, HF), _row_spec(_BE, 16)],
        out_shape=[jax.ShapeDtypeStruct((e, HF), jnp.bfloat16),
                   jax.ShapeDtypeStruct((e, HF), jnp.bfloat16),
                   jax.ShapeDtypeStruct((e, 16), f32)],
    )(g, g, eh, ec, we, emb, wih, whh, eb, wmm)


_NW = 32
_GCH = 8


def _sc_scatter_gather(q, dst2d, zeros, ps, pd, src2d):
    n = zeros.shape[0]
    e = q.shape[0]
    n_rows = dst2d.shape[0]
    srows_per_w = n_rows // _NW
    qrows_per_w = srows_per_w * _IW
    grows_per_w = n_rows // (_NW // 2)
    gout_per_w = grows_per_w * _IW
    n_ch = grows_per_w // _GCH
    nps = n // 16
    mesh = plsc.VectorSubcoreMesh(core_axis_name="c", subcore_axis_name="s")

    @functools.partial(
        pl.kernel, mesh=mesh,
        out_type=[jax.ShapeDtypeStruct((2, n, 16), jnp.float32),
                  jax.ShapeDtypeStruct((2 * e, 20), jnp.float32)],
        compiler_params=pltpu.CompilerParams(use_tc_tiling_on_sc=False),
        scratch_types=[pltpu.VMEM((srows_per_w, _IW), jnp.int32),
                       pltpu.VMEM((qrows_per_w, 16), jnp.float32),
                       pltpu.VMEM_SHARED((n, 16), jnp.float32),
                       pltpu.VMEM((grows_per_w, _IW), jnp.int32),
                       pltpu.VMEM((_GCH * _IW, 20), jnp.float32),
                       pltpu.SemaphoreType.DMA])
    def k(q_hbm, dst_hbm, z_hbm, ps_hbm, pd_hbm, src_hbm,
          sp_hbm, g_hbm, dst_v, q_v, acc, idx_v, rows_v, sem):
        cid = lax.axis_index("c")
        sid = lax.axis_index("s")
        wid = sid * 2 + cid
        lds = [pltpu.async_copy(z_hbm.at[pl.ds(sid * nps, nps)],
                                acc.at[pl.ds(sid * nps, nps)], sem),
               pltpu.async_copy(dst_hbm.at[pl.ds(wid * srows_per_w,
                                                 srows_per_w)], dst_v, sem),
               pltpu.async_copy(q_hbm.at[pl.ds(wid * qrows_per_w,
                                               qrows_per_w)], q_v, sem)]
        for c in lds:
            c.wait()
        plsc.subcore_barrier()

        def sbody(t, carry):
            cps = []
            for j in range(8):
                r = t * 8 + j
                cps.append(pltpu.async_copy(q_v.at[pl.ds(r * _IW, _IW)],
                                            acc.at[dst_v.at[r]], sem,
                                            add=True))
            for c in cps:
                c.wait()
            return carry

        lax.fori_loop(0, srows_per_w // 8, sbody, 0)
        plsc.subcore_barrier()
        pltpu.sync_copy(acc.at[pl.ds(sid * nps, nps)],
                        sp_hbm.at[cid, pl.ds(sid * nps, nps)])

        obase = wid * gout_per_w

        def gather_from(tbl_hbm, ix_hbm, ibase):
            pltpu.sync_copy(ix_hbm.at[pl.ds(ibase, grows_per_w)], idx_v)

            def chunk(t, carry):
                cps = []
                for j in range(_GCH):
                    cps.append(pltpu.async_copy(
                        tbl_hbm.at[idx_v.at[t * _GCH + j]],
                        rows_v.at[pl.ds(j * _IW, _IW)], sem))
                for c in cps:
                    c.wait()
                pltpu.sync_copy(
                    rows_v,
                    g_hbm.at[pl.ds(obase + t * (_GCH * _IW), _GCH * _IW)])
                return carry

            lax.fori_loop(0, n_ch, chunk, 0)

        @pl.when(wid < 16)
        def _():
            gather_from(ps_hbm, src_hbm, wid * grows_per_w)

        @pl.when(wid >= 16)
        def _():
            gather_from(pd_hbm, dst_hbm, (wid - 16) * grows_per_w)

    return k(q, dst2d, zeros, ps, pd, src2d)


def _sc_scatter(q, dst2d, zeros):
    n = zeros.shape[0]
    n_rows = dst2d.shape[0]
    rows_per_w = n_rows // _NW
    qrows_per_w = rows_per_w * _IW
    nps = n // 16
    mesh = plsc.VectorSubcoreMesh(core_axis_name="c", subcore_axis_name="s")

    @functools.partial(
        pl.kernel, mesh=mesh,
        out_type=jax.ShapeDtypeStruct((2, n, 16), jnp.float32),
        compiler_params=pltpu.CompilerParams(use_tc_tiling_on_sc=False),
        scratch_types=[pltpu.VMEM((rows_per_w, _IW), jnp.int32),
                       pltpu.VMEM((qrows_per_w, 16), jnp.float32),
                       pltpu.VMEM_SHARED((n, 16), jnp.float32),
                       pltpu.SemaphoreType.DMA])
    def k(q_hbm, dst_hbm, z_hbm, out_hbm, dst_v, q_v, acc, sem):
        cid = lax.axis_index("c")
        sid = lax.axis_index("s")
        wid = sid * 2 + cid
        lds = [pltpu.async_copy(z_hbm.at[pl.ds(sid * nps, nps)],
                                acc.at[pl.ds(sid * nps, nps)], sem),
               pltpu.async_copy(dst_hbm.at[pl.ds(wid * rows_per_w,
                                                 rows_per_w)], dst_v, sem),
               pltpu.async_copy(q_hbm.at[pl.ds(wid * qrows_per_w,
                                               qrows_per_w)], q_v, sem)]
        for c in lds:
            c.wait()
        plsc.subcore_barrier()

        def body(t, carry):
            cps = []
            for j in range(8):
                r = t * 8 + j
                cps.append(pltpu.async_copy(q_v.at[pl.ds(r * _IW, _IW)],
                                            acc.at[dst_v.at[r]], sem,
                                            add=True))
            for c in cps:
                c.wait()
            return carry

        lax.fori_loop(0, rows_per_w // 8, body, 0)
        plsc.subcore_barrier()
        pltpu.sync_copy(acc.at[pl.ds(sid * nps, nps)],
                        out_hbm.at[cid, pl.ds(sid * nps, nps)])

    return k(q, dst2d, zeros)


def kernel(node_feat, edge_feat, edge_index, fc1_W, fc1_b,
           n_Wih, n_Whh, n_bih, n_bhh,
           e_Wih, e_Whh, e_bih, e_bhh,
           node_mpn_W, node_mpn_b, edge_mpn_W, edge_mpn_b,
           pred_W, pred_b):
    n = node_feat.shape[0]
    e = edge_feat.shape[0]
    niter = pred_b.shape[0]
    f32 = jnp.float32

    src = edge_index[0]
    dst = edge_index[1]
    src2d = src.reshape(-1, _IW)
    dst2d = dst.reshape(-1, _IW)

    nb = (n_bih + n_bhh).reshape(1, -1)
    eb = (e_bih + e_bhh).reshape(1, -1)
    emb = edge_mpn_b.reshape(1, -1)
    nmb = node_mpn_b.reshape(1, -1)
    f1b = fc1_b.reshape(1, -1)
    ws = edge_mpn_W[:, :HF]
    we = edge_mpn_W[:, HF:2 * HF]
    wd = edge_mpn_W[:, 2 * HF:]
    wmm = node_mpn_W[:, :HF]
    wnh = node_mpn_W[:, HF:]
    bf16 = jnp.bfloat16
    we_b = we.astype(bf16)
    e_Wih_b = e_Wih.astype(bf16)
    e_Whh_b = e_Whh.astype(bf16)
    wmm_b = wmm.astype(bf16)

    n_pad = ((n + 127) // 128) * 128
    zeros_n = jnp.zeros((n_pad, 16), f32)
    deg_p = _sc_scatter(jnp.ones((e, 16), f32), dst2d, zeros_n)
    deg = deg_p[0, :n, :1] + deg_p[1, :n, :1]

    nh, nc, out, ps, pd = _node0_call(
        node_feat, n_Wih, nb, pred_W[0], pred_b[0].reshape(1, -1), ws, wd)
    eh, ec, qe = _edge0_call(edge_feat, fc1_W, f1b, e_Wih, eb, wmm)

    for it in range(1, niter):
        last = it == niter - 1
        if last:
            sp = _sc_scatter(qe, dst2d, zeros_n)
        else:
            sp, g = _sc_scatter_gather(qe, dst2d, zeros_n, ps, pd, src2d)
        nh, nc, out, ps, pd = _node_call(
            sp, deg, nh, nc, out, wnh, nmb, n_Wih, n_Whh, nb,
            pred_W[it], pred_b[it].reshape(1, -1), ws, wd)
        if not last:
            eh, ec, qe = _edge_call(g, eh, ec, we_b, emb, e_Wih_b, e_Whh_b,
                                    eb, wmm_b)

    return out


# speedup vs baseline: 1.0374x; 1.0275x over previous
import functools

import jax
import jax.numpy as jnp
from jax import lax
from jax.experimental import pallas as pl
from jax.experimental.pallas import tpu as pltpu
from jax.experimental.pallas import tpu_sc as plsc

HF = 128
_BN = 2000
_BE = 8000
_IW = 125


def _mm_t(x, w):
    return lax.dot_general(x, w, (((1,), (1,)), ((), ())),
                           preferred_element_type=jnp.float32)


def _lstm_math(gates, c):
    i, f, g, o = jnp.split(gates, 4, axis=1)
    c2 = jax.nn.sigmoid(f) * c + jax.nn.sigmoid(i) * jnp.tanh(g)
    h2 = jax.nn.sigmoid(o) * jnp.tanh(c2)
    return h2, c2


def _node0_body(x_ref, wih_ref, nb_ref, pw_ref, pb_ref, ws_ref, wd_ref,
                nh_ref, nc_ref, out_ref, ps_ref, pd_ref):
    gates = _mm_t(x_ref[...], wih_ref[...]) + nb_ref[...]
    h2, c2 = _lstm_math(gates, 0.0)
    nh_ref[...] = h2
    nc_ref[...] = c2
    out_ref[...] = _mm_t(h2, pw_ref[...]) + pb_ref[...]
    ps_ref[...] = _mm_t(h2, ws_ref[...])
    pd_ref[...] = _mm_t(h2, wd_ref[...])


def _node_body(s0_ref, s1_ref, deg_ref, nh_ref, nc_ref, out_ref,
               wnh_ref, nmb_ref, wih_ref, whh_ref, nb_ref, pw_ref, pb_ref,
               ws_ref, wd_ref,
               nh2_ref, nc2_ref, out2_ref, ps_ref, pd_ref):
    nh = nh_ref[...]
    s = (s0_ref[0] + s1_ref[0]) / jnp.maximum(deg_ref[...], 1.0)
    x = jax.nn.leaky_relu(s + _mm_t(nh, wnh_ref[...]) + nmb_ref[...], 0.01)
    gates = (_mm_t(x, wih_ref[...]) + _mm_t(nh, whh_ref[...]) + nb_ref[...])
    h2, c2 = _lstm_math(gates, nc_ref[...])
    nh2_ref[...] = h2
    nc2_ref[...] = c2
    out2_ref[...] = out_ref[...] + _mm_t(h2, pw_ref[...]) + pb_ref[...]
    ps_ref[...] = _mm_t(h2, ws_ref[...])
    pd_ref[...] = _mm_t(h2, wd_ref[...])


def _edge0_body(ef_ref, f1w_ref, f1b_ref, wih_ref, eb_ref, wmm_ref,
                eh_ref, ec_ref, qe_ref):
    x0 = _mm_t(ef_ref[...], f1w_ref[...]) + f1b_ref[...]
    gates = _mm_t(x0, wih_ref[...]) + eb_ref[...]
    h2, c2 = _lstm_math(gates, 0.0)
    eh_ref[...] = h2.astype(jnp.bfloat16)
    ec_ref[...] = c2.astype(jnp.bfloat16)
    qe_ref[...] = _mm_t(h2, wmm_ref[...])


def _edge_body(gs_ref, gd_ref, eh_ref, ec_ref,
               we_ref, emb_ref, wih_ref, whh_ref, eb_ref, wmm_ref,
               eh2_ref, ec2_ref, qe_ref):
    ehb = eh_ref[...]
    x = jax.nn.leaky_relu(
        gs_ref[...] + gd_ref[...] + _mm_t(ehb, we_ref[...]) + emb_ref[...],
        0.01)
    gates = (_mm_t(x.astype(jnp.bfloat16), wih_ref[...]) +
             _mm_t(ehb, whh_ref[...]) + eb_ref[...])
    h2, c2 = _lstm_math(gates, ec_ref[...].astype(jnp.float32))
    h2b = h2.astype(jnp.bfloat16)
    eh2_ref[...] = h2b
    ec2_ref[...] = c2.astype(jnp.bfloat16)
    qe_ref[...] = _mm_t(h2b, wmm_ref[...])


def _full_spec(a):
    nd = a.ndim
    return pl.BlockSpec(a.shape, lambda i, _nd=nd: (0,) * _nd)


def _row_spec(block_rows, cols, off=0):
    return pl.BlockSpec((block_rows, cols), lambda i, _o=off: (i + _o, 0))


def _node0_call(node_feat, wih, nb, pw, pb, ws, wd):
    n = node_feat.shape[0]
    f32 = jnp.float32
    return pl.pallas_call(
        _node0_body,
        grid=(n // _BN,),
        in_specs=[_row_spec(_BN, node_feat.shape[1])] +
                 [_full_spec(a) for a in (wih, nb, pw, pb, ws, wd)],
        out_specs=[_row_spec(_BN, HF), _row_spec(_BN, HF), _row_spec(_BN, 4),
                   _row_spec(_BN, 20), _row_spec(_BN, 20)],
        out_shape=[jax.ShapeDtypeStruct((n, HF), f32),
                   jax.ShapeDtypeStruct((n, HF), f32),
                   jax.ShapeDtypeStruct((n, 4), f32),
                   jax.ShapeDtypeStruct((n, 20), f32),
                   jax.ShapeDtypeStruct((n, 20), f32)],
    )(node_feat, wih, nb, pw, pb, ws, wd)


def _node_call(sp, deg, nh, nc, out, wnh, nmb, wih, whh, nb, pw, pb,
               ws, wd):
    n = nh.shape[0]
    f32 = jnp.float32
    sp_spec0 = pl.BlockSpec((1, _BN, 16), lambda i: (0, i, 0))
    sp_spec1 = pl.BlockSpec((1, _BN, 16), lambda i: (1, i, 0))
    return pl.pallas_call(
        _node_body,
        grid=(n // _BN,),
        in_specs=[sp_spec0, sp_spec1, _row_spec(_BN, 1),
                  _row_spec(_BN, HF), _row_spec(_BN, HF), _row_spec(_BN, 4)] +
                 [_full_spec(a) for a in (wnh, nmb, wih, whh, nb, pw, pb,
                                          ws, wd)],
        out_specs=[_row_spec(_BN, HF), _row_spec(_BN, HF), _row_spec(_BN, 4),
                   _row_spec(_BN, 20), _row_spec(_BN, 20)],
        out_shape=[jax.ShapeDtypeStruct((n, HF), f32),
                   jax.ShapeDtypeStruct((n, HF), f32),
                   jax.ShapeDtypeStruct((n, 4), f32),
                   jax.ShapeDtypeStruct((n, 20), f32),
                   jax.ShapeDtypeStruct((n, 20), f32)],
    )(sp, sp, deg, nh, nc, out, wnh, nmb, wih, whh, nb, pw, pb, ws, wd)


def _edge0_call(edge_feat, f1w, f1b, wih, eb, wmm):
    e = edge_feat.shape[0]
    f32 = jnp.float32
    return pl.pallas_call(
        _edge0_body,
        grid=(e // _BE,),
        in_specs=[_row_spec(_BE, edge_feat.shape[1])] +
                 [_full_spec(a) for a in (f1w, f1b, wih, eb, wmm)],
        out_specs=[_row_spec(_BE, HF), _row_spec(_BE, HF), _row_spec(_BE, 16)],
        out_shape=[jax.ShapeDtypeStruct((e, HF), jnp.bfloat16),
                   jax.ShapeDtypeStruct((e, HF), jnp.bfloat16),
                   jax.ShapeDtypeStruct((e, 16), f32)],
    )(edge_feat, f1w, f1b, wih, eb, wmm)


def _edge_call(g, eh, ec, we, emb, wih, whh, eb, wmm):
    e = eh.shape[0]
    f32 = jnp.float32
    nbe = e // _BE
    return pl.pallas_call(
        _edge_body,
        grid=(nbe,),
        in_specs=[_row_spec(_BE, 20), _row_spec(_BE, 20, off=nbe),
                  _row_spec(_BE, HF), _row_spec(_BE, HF)] +
                 [_full_spec(a) for a in (we, emb, wih, whh, eb, wmm)],
        out_specs=[_row_spec(_BE, HF), _row_spec(_BE, HF), _row_spec(_BE, 16)],
        out_shape=[jax.ShapeDtypeStruct((e, HF), jnp.bfloat16),
                   jax.ShapeDtypeStruct((e, HF), jnp.bfloat16),
                   jax.ShapeDtypeStruct((e, 16), f32)],
    )(g, g, eh, ec, we, emb, wih, whh, eb, wmm)


_NW = 32
_GCH = 8


def _sc_scatter_gather(q, dst2d, zeros, ps, pd, src2d):
    n = zeros.shape[0]
    e = q.shape[0]
    n_rows = dst2d.shape[0]
    srows_per_w = n_rows // _NW
    qrows_per_w = srows_per_w * _IW
    grows_per_w = n_rows // (_NW // 2)
    gout_per_w = grows_per_w * _IW
    n_ch = grows_per_w // _GCH
    nps = n // 16
    mesh = plsc.VectorSubcoreMesh(core_axis_name="c", subcore_axis_name="s")

    @functools.partial(
        pl.kernel, mesh=mesh,
        out_type=[jax.ShapeDtypeStruct((2, n, 16), jnp.float32),
                  jax.ShapeDtypeStruct((2 * e, 20), jnp.float32)],
        compiler_params=pltpu.CompilerParams(use_tc_tiling_on_sc=False),
        scratch_types=[pltpu.VMEM((srows_per_w, _IW), jnp.int32),
                       pltpu.VMEM((qrows_per_w, 16), jnp.float32),
                       pltpu.VMEM_SHARED((n, 16), jnp.float32),
                       pltpu.VMEM((grows_per_w, _IW), jnp.int32),
                       pltpu.VMEM((_GCH * _IW, 20), jnp.float32),
                       pltpu.SemaphoreType.DMA])
    def k(q_hbm, dst_hbm, z_hbm, ps_hbm, pd_hbm, src_hbm,
          sp_hbm, g_hbm, dst_v, q_v, acc, idx_v, rows_v, sem):
        cid = lax.axis_index("c")
        sid = lax.axis_index("s")
        wid = sid * 2 + cid
        lds = [pltpu.async_copy(z_hbm.at[pl.ds(sid * nps, nps)],
                                acc.at[pl.ds(sid * nps, nps)], sem),
               pltpu.async_copy(dst_hbm.at[pl.ds(wid * srows_per_w,
                                                 srows_per_w)], dst_v, sem),
               pltpu.async_copy(q_hbm.at[pl.ds(wid * qrows_per_w,
                                               qrows_per_w)], q_v, sem)]
        for c in lds:
            c.wait()
        plsc.subcore_barrier()

        def sbody(t, carry):
            cps = []
            for j in range(8):
                r = t * 8 + j
                cps.append(pltpu.async_copy(q_v.at[pl.ds(r * _IW, _IW)],
                                            acc.at[dst_v.at[r]], sem,
                                            add=True))
            for c in cps:
                c.wait()
            return carry

        lax.fori_loop(0, srows_per_w // 8, sbody, 0)
        plsc.subcore_barrier()
        pltpu.sync_copy(acc.at[pl.ds(sid * nps, nps)],
                        sp_hbm.at[cid, pl.ds(sid * nps, nps)])

        obase = wid * gout_per_w

        def gather_from(tbl_hbm, ix_hbm, ibase):
            pltpu.sync_copy(ix_hbm.at[pl.ds(ibase, grows_per_w)], idx_v)

            def chunk(t, carry):
                cps = []
                for j in range(_GCH):
                    cps.append(pltpu.async_copy(
                        tbl_hbm.at[idx_v.at[t * _GCH + j]],
                        rows_v.at[pl.ds(j * _IW, _IW)], sem))
                for c in cps:
                    c.wait()
                pltpu.sync_copy(
                    rows_v,
                    g_hbm.at[pl.ds(obase + t * (_GCH * _IW), _GCH * _IW)])
                return carry

            lax.fori_loop(0, n_ch, chunk, 0)

        @pl.when(wid < 16)
        def _():
            gather_from(ps_hbm, src_hbm, wid * grows_per_w)

        @pl.when(wid >= 16)
        def _():
            gather_from(pd_hbm, dst_hbm, (wid - 16) * grows_per_w)

    return k(q, dst2d, zeros, ps, pd, src2d)


def _sc_scatter(q, dst2d, zeros):
    n = zeros.shape[0]
    n_rows = dst2d.shape[0]
    rows_per_w = n_rows // _NW
    qrows_per_w = rows_per_w * _IW
    nps = n // 16
    mesh = plsc.VectorSubcoreMesh(core_axis_name="c", subcore_axis_name="s")

    @functools.partial(
        pl.kernel, mesh=mesh,
        out_type=jax.ShapeDtypeStruct((2, n, 16), jnp.float32),
        compiler_params=pltpu.CompilerParams(use_tc_tiling_on_sc=False),
        scratch_types=[pltpu.VMEM((rows_per_w, _IW), jnp.int32),
                       pltpu.VMEM((qrows_per_w, 16), jnp.float32),
                       pltpu.VMEM_SHARED((n, 16), jnp.float32),
                       pltpu.SemaphoreType.DMA])
    def k(q_hbm, dst_hbm, z_hbm, out_hbm, dst_v, q_v, acc, sem):
        cid = lax.axis_index("c")
        sid = lax.axis_index("s")
        wid = sid * 2 + cid
        lds = [pltpu.async_copy(z_hbm.at[pl.ds(sid * nps, nps)],
                                acc.at[pl.ds(sid * nps, nps)], sem),
               pltpu.async_copy(dst_hbm.at[pl.ds(wid * rows_per_w,
                                                 rows_per_w)], dst_v, sem),
               pltpu.async_copy(q_hbm.at[pl.ds(wid * qrows_per_w,
                                               qrows_per_w)], q_v, sem)]
        for c in lds:
            c.wait()
        plsc.subcore_barrier()

        def body(t, carry):
            cps = []
            for j in range(8):
                r = t * 8 + j
                cps.append(pltpu.async_copy(q_v.at[pl.ds(r * _IW, _IW)],
                                            acc.at[dst_v.at[r]], sem,
                                            add=True))
            for c in cps:
                c.wait()
            return carry

        lax.fori_loop(0, rows_per_w // 8, body, 0)
        plsc.subcore_barrier()
        pltpu.sync_copy(acc.at[pl.ds(sid * nps, nps)],
                        out_hbm.at[cid, pl.ds(sid * nps, nps)])

    return k(q, dst2d, zeros)


def kernel(node_feat, edge_feat, edge_index, fc1_W, fc1_b,
           n_Wih, n_Whh, n_bih, n_bhh,
           e_Wih, e_Whh, e_bih, e_bhh,
           node_mpn_W, node_mpn_b, edge_mpn_W, edge_mpn_b,
           pred_W, pred_b):
    n = node_feat.shape[0]
    e = edge_feat.shape[0]
    niter = pred_b.shape[0]
    f32 = jnp.float32

    src = edge_index[0]
    dst = edge_index[1]
    src2d = src.reshape(-1, _IW)
    dst2d = dst.reshape(-1, _IW)

    nb = (n_bih + n_bhh).reshape(1, -1)
    eb = (e_bih + e_bhh).reshape(1, -1)
    emb = edge_mpn_b.reshape(1, -1)
    nmb = node_mpn_b.reshape(1, -1)
    f1b = fc1_b.reshape(1, -1)
    ws = edge_mpn_W[:, :HF]
    we = edge_mpn_W[:, HF:2 * HF]
    wd = edge_mpn_W[:, 2 * HF:]
    wmm = node_mpn_W[:, :HF]
    wnh = node_mpn_W[:, HF:]
    bf16 = jnp.bfloat16
    we_b = we.astype(bf16)
    e_Wih_b = e_Wih.astype(bf16)
    e_Whh_b = e_Whh.astype(bf16)
    wmm_b = wmm.astype(bf16)

    n_pad = ((n + 127) // 128) * 128
    zeros_n = jnp.zeros((n_pad, 16), f32)
    deg_p = _sc_scatter(jnp.ones((e, 16), f32), dst2d, zeros_n)
    deg = deg_p[0, :n, :1] + deg_p[1, :n, :1]

    nh, nc, out, ps, pd = _node0_call(
        node_feat, n_Wih, nb, pred_W[0], pred_b[0].reshape(1, -1), ws, wd)
    eh, ec, qe = _edge0_call(edge_feat, fc1_W, f1b, e_Wih, eb, wmm)

    for it in range(1, niter):
        last = it == niter - 1
        if last:
            sp = _sc_scatter(qe, dst2d, zeros_n)
        else:
            sp, g = _sc_scatter_gather(qe, dst2d, zeros_n, ps, pd, src2d)
        nh, nc, out, ps, pd = _node_call(
            sp, deg, nh, nc, out, wnh, nmb, n_Wih, n_Whh, nb,
            pred_W[it], pred_b[it].reshape(1, -1), ws, wd)
        if not last:
            eh, ec, qe = _edge_call(g, eh, ec, we_b, emb, e_Wih_b, e_Whh_b,
                                    eb, wmm_b)

    return out


# speedup vs baseline: 1.0407x; 1.0032x over previous
import functools

import jax
import jax.numpy as jnp
from jax import lax
from jax.experimental import pallas as pl
from jax.experimental.pallas import tpu as pltpu
from jax.experimental.pallas import tpu_sc as plsc

HF = 128
_BN = 2000
_BE = 8000
_IW = 125


def _mm_t(x, w):
    return lax.dot_general(x, w, (((1,), (1,)), ((), ())),
                           preferred_element_type=jnp.float32)


def _lstm_math(gates, c):
    i, f, g, o = jnp.split(gates, 4, axis=1)
    c2 = jax.nn.sigmoid(f) * c + jax.nn.sigmoid(i) * jnp.tanh(g)
    h2 = jax.nn.sigmoid(o) * jnp.tanh(c2)
    return h2, c2


def _node0_body(x_ref, wih_ref, nb_ref, pw_ref, pb_ref, ws_ref, wd_ref,
                nh_ref, nc_ref, out_ref, ps_ref, pd_ref):
    gates = _mm_t(x_ref[...], wih_ref[...]) + nb_ref[...]
    h2, c2 = _lstm_math(gates, 0.0)
    nh_ref[...] = h2
    nc_ref[...] = c2
    out_ref[...] = _mm_t(h2, pw_ref[...]) + pb_ref[...]
    ps_ref[...] = _mm_t(h2, ws_ref[...])
    pd_ref[...] = _mm_t(h2, wd_ref[...])


def _node_body(s0_ref, s1_ref, deg_ref, nh_ref, nc_ref, out_ref,
               wnh_ref, nmb_ref, wih_ref, whh_ref, nb_ref, pw_ref, pb_ref,
               ws_ref, wd_ref,
               nh2_ref, nc2_ref, out2_ref, ps_ref, pd_ref):
    nh = nh_ref[...]
    s = ((s0_ref[0] + s1_ref[0]).astype(jnp.float32) /
         jnp.maximum(deg_ref[...], 1.0))
    x = jax.nn.leaky_relu(s + _mm_t(nh, wnh_ref[...]) + nmb_ref[...], 0.01)
    gates = (_mm_t(x, wih_ref[...]) + _mm_t(nh, whh_ref[...]) + nb_ref[...])
    h2, c2 = _lstm_math(gates, nc_ref[...])
    nh2_ref[...] = h2
    nc2_ref[...] = c2
    out2_ref[...] = out_ref[...] + _mm_t(h2, pw_ref[...]) + pb_ref[...]
    ps_ref[...] = _mm_t(h2, ws_ref[...])
    pd_ref[...] = _mm_t(h2, wd_ref[...])


def _edge0_body(ef_ref, f1w_ref, f1b_ref, wih_ref, eb_ref, wmm_ref,
                eh_ref, ec_ref, qe_ref):
    x0 = _mm_t(ef_ref[...], f1w_ref[...]) + f1b_ref[...]
    gates = _mm_t(x0, wih_ref[...]) + eb_ref[...]
    h2, c2 = _lstm_math(gates, 0.0)
    eh_ref[...] = h2.astype(jnp.bfloat16)
    ec_ref[...] = c2.astype(jnp.bfloat16)
    qe_ref[...] = _mm_t(h2, wmm_ref[...]).astype(jnp.bfloat16)


def _edge_body(gs_ref, gd_ref, eh_ref, ec_ref,
               we_ref, emb_ref, wih_ref, whh_ref, eb_ref, wmm_ref,
               eh2_ref, ec2_ref, qe_ref):
    ehb = eh_ref[...]
    x = jax.nn.leaky_relu(
        gs_ref[...] + gd_ref[...] + _mm_t(ehb, we_ref[...]) + emb_ref[...],
        0.01)
    gates = (_mm_t(x.astype(jnp.bfloat16), wih_ref[...]) +
             _mm_t(ehb, whh_ref[...]) + eb_ref[...])
    h2, c2 = _lstm_math(gates, ec_ref[...].astype(jnp.float32))
    h2b = h2.astype(jnp.bfloat16)
    eh2_ref[...] = h2b
    ec2_ref[...] = c2.astype(jnp.bfloat16)
    qe_ref[...] = _mm_t(h2b, wmm_ref[...]).astype(jnp.bfloat16)


def _full_spec(a):
    nd = a.ndim
    return pl.BlockSpec(a.shape, lambda i, _nd=nd: (0,) * _nd)


def _row_spec(block_rows, cols, off=0):
    return pl.BlockSpec((block_rows, cols), lambda i, _o=off: (i + _o, 0))


def _node0_call(node_feat, wih, nb, pw, pb, ws, wd):
    n = node_feat.shape[0]
    f32 = jnp.float32
    return pl.pallas_call(
        _node0_body,
        grid=(n // _BN,),
        in_specs=[_row_spec(_BN, node_feat.shape[1])] +
                 [_full_spec(a) for a in (wih, nb, pw, pb, ws, wd)],
        out_specs=[_row_spec(_BN, HF), _row_spec(_BN, HF), _row_spec(_BN, 4),
                   _row_spec(_BN, 20), _row_spec(_BN, 20)],
        out_shape=[jax.ShapeDtypeStruct((n, HF), f32),
                   jax.ShapeDtypeStruct((n, HF), f32),
                   jax.ShapeDtypeStruct((n, 4), f32),
                   jax.ShapeDtypeStruct((n, 20), f32),
                   jax.ShapeDtypeStruct((n, 20), f32)],
    )(node_feat, wih, nb, pw, pb, ws, wd)


def _node_call(sp, deg, nh, nc, out, wnh, nmb, wih, whh, nb, pw, pb,
               ws, wd):
    n = nh.shape[0]
    f32 = jnp.float32
    sp_spec0 = pl.BlockSpec((1, _BN, 16), lambda i: (0, i, 0))
    sp_spec1 = pl.BlockSpec((1, _BN, 16), lambda i: (1, i, 0))
    return pl.pallas_call(
        _node_body,
        grid=(n // _BN,),
        in_specs=[sp_spec0, sp_spec1, _row_spec(_BN, 1),
                  _row_spec(_BN, HF), _row_spec(_BN, HF), _row_spec(_BN, 4)] +
                 [_full_spec(a) for a in (wnh, nmb, wih, whh, nb, pw, pb,
                                          ws, wd)],
        out_specs=[_row_spec(_BN, HF), _row_spec(_BN, HF), _row_spec(_BN, 4),
                   _row_spec(_BN, 20), _row_spec(_BN, 20)],
        out_shape=[jax.ShapeDtypeStruct((n, HF), f32),
                   jax.ShapeDtypeStruct((n, HF), f32),
                   jax.ShapeDtypeStruct((n, 4), f32),
                   jax.ShapeDtypeStruct((n, 20), f32),
                   jax.ShapeDtypeStruct((n, 20), f32)],
    )(sp, sp, deg, nh, nc, out, wnh, nmb, wih, whh, nb, pw, pb, ws, wd)


def _edge0_call(edge_feat, f1w, f1b, wih, eb, wmm):
    e = edge_feat.shape[0]
    f32 = jnp.float32
    return pl.pallas_call(
        _edge0_body,
        grid=(e // _BE,),
        in_specs=[_row_spec(_BE, edge_feat.shape[1])] +
                 [_full_spec(a) for a in (f1w, f1b, wih, eb, wmm)],
        out_specs=[_row_spec(_BE, HF), _row_spec(_BE, HF), _row_spec(_BE, 16)],
        out_shape=[jax.ShapeDtypeStruct((e, HF), jnp.bfloat16),
                   jax.ShapeDtypeStruct((e, HF), jnp.bfloat16),
                   jax.ShapeDtypeStruct((e, 16), jnp.bfloat16)],
    )(edge_feat, f1w, f1b, wih, eb, wmm)


def _edge_call(g, eh, ec, we, emb, wih, whh, eb, wmm):
    e = eh.shape[0]
    f32 = jnp.float32
    nbe = e // _BE
    return pl.pallas_call(
        _edge_body,
        grid=(nbe,),
        in_specs=[_row_spec(_BE, 20), _row_spec(_BE, 20, off=nbe),
                  _row_spec(_BE, HF), _row_spec(_BE, HF)] +
                 [_full_spec(a) for a in (we, emb, wih, whh, eb, wmm)],
        out_specs=[_row_spec(_BE, HF), _row_spec(_BE, HF), _row_spec(_BE, 16)],
        out_shape=[jax.ShapeDtypeStruct((e, HF), jnp.bfloat16),
                   jax.ShapeDtypeStruct((e, HF), jnp.bfloat16),
                   jax.ShapeDtypeStruct((e, 16), jnp.bfloat16)],
    )(g, g, eh, ec, we, emb, wih, whh, eb, wmm)


_NW = 32
_GCH = 8


def _sc_scatter_gather(q, dst2d, zeros, ps, pd, src2d):
    n = zeros.shape[0]
    e = q.shape[0]
    n_rows = dst2d.shape[0]
    srows_per_w = n_rows // _NW
    qrows_per_w = srows_per_w * _IW
    grows_per_w = n_rows // (_NW // 2)
    gout_per_w = grows_per_w * _IW
    n_ch = grows_per_w // _GCH
    nps = n // 16
    mesh = plsc.VectorSubcoreMesh(core_axis_name="c", subcore_axis_name="s")

    @functools.partial(
        pl.kernel, mesh=mesh,
        out_type=[jax.ShapeDtypeStruct((2, n, 16), jnp.bfloat16),
                  jax.ShapeDtypeStruct((2 * e, 20), jnp.float32)],
        compiler_params=pltpu.CompilerParams(use_tc_tiling_on_sc=False),
        scratch_types=[pltpu.VMEM((srows_per_w, _IW), jnp.int32),
                       pltpu.VMEM((qrows_per_w, 16), jnp.bfloat16),
                       pltpu.VMEM_SHARED((n, 16), jnp.bfloat16),
                       pltpu.VMEM((grows_per_w, _IW), jnp.int32),
                       pltpu.VMEM((_GCH * _IW, 20), jnp.float32),
                       pltpu.SemaphoreType.DMA])
    def k(q_hbm, dst_hbm, z_hbm, ps_hbm, pd_hbm, src_hbm,
          sp_hbm, g_hbm, dst_v, q_v, acc, idx_v, rows_v, sem):
        cid = lax.axis_index("c")
        sid = lax.axis_index("s")
        wid = sid * 2 + cid
        lds = [pltpu.async_copy(z_hbm.at[pl.ds(sid * nps, nps)],
                                acc.at[pl.ds(sid * nps, nps)], sem),
               pltpu.async_copy(dst_hbm.at[pl.ds(wid * srows_per_w,
                                                 srows_per_w)], dst_v, sem),
               pltpu.async_copy(q_hbm.at[pl.ds(wid * qrows_per_w,
                                               qrows_per_w)], q_v, sem)]
        for c in lds:
            c.wait()
        plsc.subcore_barrier()

        def sbody(t, carry):
            cps = []
            for j in range(8):
                r = t * 8 + j
                cps.append(pltpu.async_copy(q_v.at[pl.ds(r * _IW, _IW)],
                                            acc.at[dst_v.at[r]], sem,
                                            add=True))
            for c in cps:
                c.wait()
            return carry

        lax.fori_loop(0, srows_per_w // 8, sbody, 0)
        plsc.subcore_barrier()
        pltpu.sync_copy(acc.at[pl.ds(sid * nps, nps)],
                        sp_hbm.at[cid, pl.ds(sid * nps, nps)])

        obase = wid * gout_per_w

        def gather_from(tbl_hbm, ix_hbm, ibase):
            pltpu.sync_copy(ix_hbm.at[pl.ds(ibase, grows_per_w)], idx_v)

            def chunk(t, carry):
                cps = []
                for j in range(_GCH):
                    cps.append(pltpu.async_copy(
                        tbl_hbm.at[idx_v.at[t * _GCH + j]],
                        rows_v.at[pl.ds(j * _IW, _IW)], sem))
                for c in cps:
                    c.wait()
                pltpu.sync_copy(
                    rows_v,
                    g_hbm.at[pl.ds(obase + t * (_GCH * _IW), _GCH * _IW)])
                return carry

            lax.fori_loop(0, n_ch, chunk, 0)

        @pl.when(wid < 16)
        def _():
            gather_from(ps_hbm, src_hbm, wid * grows_per_w)

        @pl.when(wid >= 16)
        def _():
            gather_from(pd_hbm, dst_hbm, (wid - 16) * grows_per_w)

    return k(q, dst2d, zeros, ps, pd, src2d)


def _sc_scatter(q, dst2d, zeros):
    n = zeros.shape[0]
    n_rows = dst2d.shape[0]
    rows_per_w = n_rows // _NW
    qrows_per_w = rows_per_w * _IW
    nps = n // 16
    mesh = plsc.VectorSubcoreMesh(core_axis_name="c", subcore_axis_name="s")

    @functools.partial(
        pl.kernel, mesh=mesh,
        out_type=jax.ShapeDtypeStruct((2, n, 16), jnp.bfloat16),
        compiler_params=pltpu.CompilerParams(use_tc_tiling_on_sc=False),
        scratch_types=[pltpu.VMEM((rows_per_w, _IW), jnp.int32),
                       pltpu.VMEM((qrows_per_w, 16), jnp.bfloat16),
                       pltpu.VMEM_SHARED((n, 16), jnp.bfloat16),
                       pltpu.SemaphoreType.DMA])
    def k(q_hbm, dst_hbm, z_hbm, out_hbm, dst_v, q_v, acc, sem):
        cid = lax.axis_index("c")
        sid = lax.axis_index("s")
        wid = sid * 2 + cid
        lds = [pltpu.async_copy(z_hbm.at[pl.ds(sid * nps, nps)],
                                acc.at[pl.ds(sid * nps, nps)], sem),
               pltpu.async_copy(dst_hbm.at[pl.ds(wid * rows_per_w,
                                                 rows_per_w)], dst_v, sem),
               pltpu.async_copy(q_hbm.at[pl.ds(wid * qrows_per_w,
                                               qrows_per_w)], q_v, sem)]
        for c in lds:
            c.wait()
        plsc.subcore_barrier()

        def body(t, carry):
            cps = []
            for j in range(8):
                r = t * 8 + j
                cps.append(pltpu.async_copy(q_v.at[pl.ds(r * _IW, _IW)],
                                            acc.at[dst_v.at[r]], sem,
                                            add=True))
            for c in cps:
                c.wait()
            return carry

        lax.fori_loop(0, rows_per_w // 8, body, 0)
        plsc.subcore_barrier()
        pltpu.sync_copy(acc.at[pl.ds(sid * nps, nps)],
                        out_hbm.at[cid, pl.ds(sid * nps, nps)])

    return k(q, dst2d, zeros)


def kernel(node_feat, edge_feat, edge_index, fc1_W, fc1_b,
           n_Wih, n_Whh, n_bih, n_bhh,
           e_Wih, e_Whh, e_bih, e_bhh,
           node_mpn_W, node_mpn_b, edge_mpn_W, edge_mpn_b,
           pred_W, pred_b):
    n = node_feat.shape[0]
    e = edge_feat.shape[0]
    niter = pred_b.shape[0]
    f32 = jnp.float32

    src = edge_index[0]
    dst = edge_index[1]
    src2d = src.reshape(-1, _IW)
    dst2d = dst.reshape(-1, _IW)

    nb = (n_bih + n_bhh).reshape(1, -1)
    eb = (e_bih + e_bhh).reshape(1, -1)
    emb = edge_mpn_b.reshape(1, -1)
    nmb = node_mpn_b.reshape(1, -1)
    f1b = fc1_b.reshape(1, -1)
    ws = edge_mpn_W[:, :HF]
    we = edge_mpn_W[:, HF:2 * HF]
    wd = edge_mpn_W[:, 2 * HF:]
    wmm = node_mpn_W[:, :HF]
    wnh = node_mpn_W[:, HF:]
    bf16 = jnp.bfloat16
    we_b = we.astype(bf16)
    e_Wih_b = e_Wih.astype(bf16)
    e_Whh_b = e_Whh.astype(bf16)
    wmm_b = wmm.astype(bf16)

    n_pad = ((n + 127) // 128) * 128
    zeros_n = jnp.zeros((n_pad, 16), jnp.bfloat16)
    deg_p = _sc_scatter(jnp.ones((e, 16), jnp.bfloat16), dst2d, zeros_n)
    deg = (deg_p[0, :n, :1] + deg_p[1, :n, :1]).astype(f32)

    nh, nc, out, ps, pd = _node0_call(
        node_feat, n_Wih, nb, pred_W[0], pred_b[0].reshape(1, -1), ws, wd)
    eh, ec, qe = _edge0_call(edge_feat, fc1_W, f1b, e_Wih, eb, wmm)

    for it in range(1, niter):
        last = it == niter - 1
        if last:
            sp = _sc_scatter(qe, dst2d, zeros_n)
        else:
            sp, g = _sc_scatter_gather(qe, dst2d, zeros_n, ps, pd, src2d)
        nh, nc, out, ps, pd = _node_call(
            sp, deg, nh, nc, out, wnh, nmb, n_Wih, n_Whh, nb,
            pred_W[it], pred_b[it].reshape(1, -1), ws, wd)
        if not last:
            eh, ec, qe = _edge_call(g, eh, ec, we_b, emb, e_Wih_b, e_Whh_b,
                                    eb, wmm_b)

    return out
